# baseline (speedup 1.0000x reference)
import functools

import jax
import jax.numpy as jnp
from jax import lax
from jax.experimental import pallas as pl
from jax.experimental.pallas import tpu as pltpu
from jax.experimental.pallas import tpu_sc as plsc

N = 10000
D = 128
E = 320000
NQ = 4096
NP = 10240
SENT = 10000
EP = 323584
F32 = jnp.float32
I32 = jnp.int32

MESH = plsc.VectorSubcoreMesh(
    core_axis_name="c", subcore_axis_name="s", num_cores=2, num_subcores=16)

def _ids():
    c = lax.axis_index("c")
    s = lax.axis_index("s")
    return c, s, c * 16 + s


def _fill1(ref, n, val, dtype):
    v = jnp.full((16,), val, dtype)
    def body(i, _):
        ref[pl.ds(i * 16, 16)] = v
        return 0
    lax.fori_loop(0, n // 16, body, 0)


def _fill2(ref, rows, val, dtype):
    v = jnp.full((16,), val, dtype)
    def body(i, _):
        for k in range(8):
            ref[i, pl.ds(k * 16, 16)] = v
        return 0
    lax.fori_loop(0, rows, body, 0)


def _bc16(ref, i):
    j = (i // 16) * 16
    piece = ref[pl.ds(j, 16)]
    dn = lax.GatherDimensionNumbers(
        offset_dims=(), collapsed_slice_dims=(0,), start_index_map=(0,))
    return lax.gather(piece, jnp.full((16, 1), i - j, I32), dn,
                      slice_sizes=(1,),
                      mode=lax.GatherScatterMode.PROMISE_IN_BOUNDS)


def _rsqrt(d):
    h = d * (-0.5)
    i = lax.bitcast_convert_type(d, I32)
    i = jnp.int32(0x5F3759DF) - (i >> 1)
    y = lax.bitcast_convert_type(i, F32)
    for _ in range(4):
        y = y * (1.5 + h * y * y)
    return y


def _mark_body(nodes_r, out_hbm, mark_sh, zb, ones, idxv):
    c, s, _ = _ids()
    _fill1(zb, 640, 0, I32)
    _fill1(ones, 128, 1, I32)
    pltpu.sync_copy(zb, mark_sh.at[pl.ds(640 * s, 640)])
    plsc.subcore_barrier()
    w = c * 16 + s
    pltpu.sync_copy(nodes_r.at[w], idxv)
    for b in range(2):
        pltpu.sync_copy(ones, mark_sh.at[idxv.at[b]], add=True)
    plsc.subcore_barrier()
    pltpu.sync_copy(mark_sh.at[pl.ds(640 * s, 640)],
                    out_hbm.at[pl.ds(c * NP + 640 * s, 640)])


_ph_mark = pl.kernel(
    _mark_body,
    out_type=jax.ShapeDtypeStruct((2 * NP,), I32),
    mesh=MESH,
    scratch_types=[
        pltpu.VMEM_SHARED((NP,), I32),
        pltpu.VMEM((640,), I32),
        pltpu.VMEM((128,), I32),
        pltpu.VMEM((2, 128), I32),
    ],
)


def _hop_body(prev, srcr, dstr, out_hbm,
              cur_sh, nxt_sh, zb, av, bv, cv, srcv, dstv, g0, g1,
              sem0, sem1):
    c, s, w = _ids()
    base = 640 * s
    _fill1(zb, 640, 0, I32)
    pltpu.sync_copy(prev.at[pl.ds(base, 640)], av)
    pltpu.sync_copy(prev.at[pl.ds(NP + base, 640)], bv)
    def thr(i, _):
        sl = pl.ds(i * 16, 16)
        cv[sl] = jnp.where(av[sl] + bv[sl] > 0, 1, 0).astype(I32)
        return 0
    lax.fori_loop(0, 40, thr, 0)
    pltpu.sync_copy(cv, cur_sh.at[pl.ds(base, 640)])
    pltpu.sync_copy(zb, nxt_sh.at[pl.ds(base, 640)])
    plsc.subcore_barrier()
    pltpu.sync_copy(srcr.at[w], srcv)
    pltpu.sync_copy(dstr.at[w], dstv)

    pltpu.async_copy(cur_sh.at[dstv.at[0]], g0, sem0)
    def body(jj, _):
        j0 = 2 * jj
        j1 = j0 + 1
        @pl.when(j1 < 79)
        def _():
            pltpu.async_copy(cur_sh.at[dstv.at[j1]], g1, sem1)
        pltpu.make_async_copy(cur_sh.at[dstv.at[j0]], g0, sem0).wait()
        pltpu.sync_copy(g0, nxt_sh.at[srcv.at[j0]], add=True)
        @pl.when(j0 + 2 < 79)
        def _():
            pltpu.async_copy(cur_sh.at[dstv.at[j0 + 2]], g0, sem0)
        @pl.when(j1 < 79)
        def _():
            pltpu.make_async_copy(cur_sh.at[dstv.at[j1]], g1, sem1).wait()
            pltpu.sync_copy(g1, nxt_sh.at[srcv.at[j1]], add=True)
        return 0
    lax.fori_loop(0, 40, body, 0)
    plsc.subcore_barrier()
    pltpu.sync_copy(nxt_sh.at[pl.ds(base, 640)],
                    out_hbm.at[pl.ds(c * NP + base, 640)])


_ph_hop = pl.kernel(
    _hop_body,
    out_type=jax.ShapeDtypeStruct((2 * NP,), I32),
    mesh=MESH,
    scratch_types=[
        pltpu.VMEM_SHARED((NP,), I32),
        pltpu.VMEM_SHARED((NP,), I32),
        pltpu.VMEM((640,), I32),
        pltpu.VMEM((640,), I32),
        pltpu.VMEM((640,), I32),
        pltpu.VMEM((640,), I32),
        pltpu.VMEM((79, 128), I32),
        pltpu.VMEM((79, 128), I32),
        pltpu.VMEM((128,), I32),
        pltpu.VMEM((128,), I32),
        pltpu.SemaphoreType.DMA,
        pltpu.SemaphoreType.DMA,
    ],
)


def _valid_body(markp, n1p, n2p, srcr, dstr, pk2r, degp,
                reach_sh, deg_sh, zbf, m0, m1, a1, b1, a2, b2, cv,
                srcv, dstv, pkv, rs, rd, rs1, rd1, vf,
                sem, sem1):
    c, s, w = _ids()
    base = 640 * s
    _fill1(zbf, 640, 0.0, F32)
    pltpu.sync_copy(markp.at[pl.ds(base, 640)], m0)
    pltpu.sync_copy(markp.at[pl.ds(NP + base, 640)], m1)
    pltpu.sync_copy(n1p.at[pl.ds(base, 640)], a1)
    pltpu.sync_copy(n1p.at[pl.ds(NP + base, 640)], b1)
    pltpu.sync_copy(n2p.at[pl.ds(base, 640)], a2)
    pltpu.sync_copy(n2p.at[pl.ds(NP + base, 640)], b2)
    def thr(i, _):
        sl = pl.ds(i * 16, 16)
        tot = m0[sl] + m1[sl] + a1[sl] + b1[sl] + a2[sl] + b2[sl]
        cv[sl] = jnp.where(tot > 0, 1, 0).astype(I32)
        return 0
    lax.fori_loop(0, 40, thr, 0)
    pltpu.sync_copy(cv, reach_sh.at[pl.ds(base, 640)])
    pltpu.sync_copy(zbf, deg_sh.at[pl.ds(base, 640)])
    plsc.subcore_barrier()
    pltpu.sync_copy(srcr.at[w], srcv)
    pltpu.sync_copy(dstr.at[w], dstv)

    def issue(j, rsb, rdb, semb):
        pltpu.async_copy(reach_sh.at[srcv.at[j]], rsb, semb)
        pltpu.async_copy(reach_sh.at[dstv.at[j]], rdb, semb)

    def drain(j, rsb, rdb, semb):
        pltpu.make_async_copy(reach_sh.at[srcv.at[j]], rsb, semb).wait()
        pltpu.make_async_copy(reach_sh.at[dstv.at[j]], rdb, semb).wait()

    def compute(j, rsb, rdb):
        for k in range(8):
            sl = pl.ds(k * 16, 16)
            junk = lax.iota(I32, 16) + jnp.int32(SENT + 16 * k)
            junk_pk = junk | (junk << 16)
            valid = rsb[sl] & rdb[sl]
            ok = valid > 0
            packed = srcv[j, sl] | (dstv[j, sl] << 16)
            pkv[j, sl] = jnp.where(ok, packed, junk_pk)
            vf[sl] = valid.astype(F32)
        pltpu.sync_copy(vf, deg_sh.at[srcv.at[j]], add=True)

    def body(jj, _):
        j0 = 2 * jj
        j1 = j0 + 1
        @pl.when(j1 < 79)
        def _():
            issue(j1, rs1, rd1, sem1)
        drain(j0, rs, rd, sem)
        compute(j0, rs, rd)
        @pl.when(j0 + 2 < 79)
        def _():
            issue(j0 + 2, rs, rd, sem)
        @pl.when(j1 < 79)
        def _():
            drain(j1, rs1, rd1, sem1)
            compute(j1, rs1, rd1)
        return 0
    issue(0, rs, rd, sem)
    lax.fori_loop(0, 40, body, 0)
    pltpu.sync_copy(pkv, pk2r.at[w])
    plsc.subcore_barrier()
    pltpu.sync_copy(deg_sh.at[pl.ds(640 * s, 640)],
                    degp.at[pl.ds(c * NP + 640 * s, 640)])


_ph_valid = pl.kernel(
    _valid_body,
    out_type=(
        jax.ShapeDtypeStruct((32, 79, 128), I32),
        jax.ShapeDtypeStruct((2 * NP,), F32),
    ),
    mesh=MESH,
    scratch_types=[
        pltpu.VMEM_SHARED((NP,), I32),
        pltpu.VMEM_SHARED((NP,), F32),
        pltpu.VMEM((640,), F32),
        pltpu.VMEM((640,), I32),
        pltpu.VMEM((640,), I32),
        pltpu.VMEM((640,), I32),
        pltpu.VMEM((640,), I32),
        pltpu.VMEM((640,), I32),
        pltpu.VMEM((640,), I32),
        pltpu.VMEM((640,), I32),
        pltpu.VMEM((79, 128), I32),
        pltpu.VMEM((79, 128), I32),
        pltpu.VMEM((79, 128), I32),
        pltpu.VMEM((128,), I32),
        pltpu.VMEM((128,), I32),
        pltpu.VMEM((128,), I32),
        pltpu.VMEM((128,), I32),
        pltpu.VMEM((128,), F32),
        pltpu.SemaphoreType.DMA,
        pltpu.SemaphoreType.DMA,
    ],
)


def _scale_body(degp, x_pad, deg1, dis, xs,
                dp0v, dp1v, d1v, disv, xv, xsv):
    c, s, w = _ids()
    base = 320 * w
    pltpu.sync_copy(degp.at[pl.ds(base, 320)], dp0v)
    pltpu.sync_copy(degp.at[pl.ds(NP + base, 320)], dp1v)

    def body(i, _):
        sl = pl.ds(i * 16, 16)
        d1 = dp0v[sl] + dp1v[sl]
        d1v[sl] = d1
        disv[sl] = _rsqrt(d1 + 1.0)
        return 0
    lax.fori_loop(0, 20, body, 0)
    pltpu.sync_copy(d1v, deg1.at[pl.ds(base, 320)])
    pltpu.sync_copy(disv, dis.at[pl.ds(base, 320)])

    def chunk(cc, _):
        pltpu.sync_copy(x_pad.at[pl.ds(base + 64 * cc, 64)], xv)
        def row(r, _):
            b = _bc16(disv, 64 * cc + r)
            for k in range(8):
                sl = pl.ds(k * 16, 16)
                xsv[r, sl] = b * xv[r, sl]
            return 0
        lax.fori_loop(0, 64, row, 0)
        pltpu.sync_copy(xsv, xs.at[pl.ds(base + 64 * cc, 64)])
        return 0
    lax.fori_loop(0, 5, chunk, 0)


_ph_scale = pl.kernel(
    _scale_body,
    out_type=(
        jax.ShapeDtypeStruct((NP,), F32),
        jax.ShapeDtypeStruct((NP,), F32),
        jax.ShapeDtypeStruct((NP, D), F32),
    ),
    mesh=MESH,
    scratch_types=[
        pltpu.VMEM((320,), F32),
        pltpu.VMEM((320,), F32),
        pltpu.VMEM((320,), F32),
        pltpu.VMEM((320,), F32),
        pltpu.VMEM((64, D), F32),
        pltpu.VMEM((64, D), F32),
    ],
)


def _spmm_body(T, pkr, P, acc_sh, zb2, pk,
               sc0, dc0, sc1, dc1, sc2, dc2,
               rv0, rv1, rv2,
               sg0, sg1, sg2, ss0, ss1, ss2):
    c, s, w = _ids()
    _fill2(zb2, 32, 0.0, F32)
    def zc(cc, _):
        pltpu.sync_copy(zb2, acc_sh.at[pl.ds(640 * s + 32 * cc, 32)])
        return 0
    lax.fori_loop(0, 20, zc, 0)
    plsc.subcore_barrier()
    pltpu.sync_copy(pkr.at[w], pk)

    scs = (sc0, sc1, sc2)
    dcs = (dc0, dc1, dc2)
    rvs = (rv0, rv1, rv2)
    sgs = (sg0, sg1, sg2)
    sss = (ss0, ss1, ss2)

    def unpack(j, sb, db):
        row = j // 2
        half = (j % 2) * 64
        for k in range(4):
            pkd = pk[row, pl.ds(half + k * 16, 16)]
            sl = pl.ds(k * 16, 16)
            sb[sl] = pkd & jnp.int32(0xFFFF)
            db[sl] = lax.shift_right_logical(pkd, 16)

    def issue_gather(d, j):
        unpack(j, scs[d], dcs[d])
        pltpu.async_copy(T.at[dcs[d]], rvs[d], sgs[d])

    for d in range(3):
        issue_gather(d, d)

    def body(ii, _):
        j = 3 * ii
        for d in range(3):
            jd = j + d
            @pl.when(jd < 158)
            def _():
                pltpu.make_async_copy(T.at[dcs[d]], rvs[d], sgs[d]).wait()
                pltpu.async_copy(rvs[d], acc_sh.at[scs[d]], sss[d], add=True)
        for d in range(3):
            jn = j + d + 3
            @pl.when(jn < 158)
            def _():
                pltpu.make_async_copy(rvs[d], acc_sh.at[scs[d]],
                                      sss[d]).wait()
                issue_gather(d, jn)
        return 0
    lax.fori_loop(0, 53, body, 0)
    for d in range(3):
        pltpu.make_async_copy(rvs[d], acc_sh.at[scs[d]], sss[d]).wait()
    plsc.subcore_barrier()
    pltpu.sync_copy(acc_sh.at[pl.ds(640 * s, 640)],
                    P.at[pl.ds(c * NP + 640 * s, 640)])


_ph_spmm = pl.kernel(
    _spmm_body,
    out_type=jax.ShapeDtypeStruct((2 * NP, D), F32),
    mesh=MESH,
    scratch_types=(
        [pltpu.VMEM_SHARED((NP, D), F32),
         pltpu.VMEM((32, D), F32),
         pltpu.VMEM((79, 128), I32)]
        + [pltpu.VMEM((64,), I32)] * 6
        + [pltpu.VMEM((64, D), F32)] * 3
        + [pltpu.SemaphoreType.DMA] * 6
    ),
)


def _os_body(P, xs, dis, osad, disv, p0v, p1v, xsv, ov):
    c, s, w = _ids()
    base = 320 * w
    pltpu.sync_copy(dis.at[pl.ds(base, 320)], disv)

    def chunk(cc, _):
        sl64 = pl.ds(base + 64 * cc, 64)
        pltpu.sync_copy(P.at[sl64], p0v)
        pltpu.sync_copy(P.at[pl.ds(NP + base + 64 * cc, 64)], p1v)
        pltpu.sync_copy(xs.at[sl64], xsv)
        def row(r, _):
            b = _bc16(disv, 64 * cc + r)
            b2 = b * b
            for k in range(8):
                sl = pl.ds(k * 16, 16)
                ov[r, sl] = b2 * (p0v[r, sl] + p1v[r, sl] + xsv[r, sl])
            return 0
        lax.fori_loop(0, 64, row, 0)
        pltpu.sync_copy(ov, osad.at[sl64])
        return 0
    lax.fori_loop(0, 5, chunk, 0)


_ph_os = pl.kernel(
    _os_body,
    out_type=jax.ShapeDtypeStruct((NP, D), F32),
    mesh=MESH,
    scratch_types=[
        pltpu.VMEM((320,), F32),
        pltpu.VMEM((64, D), F32),
        pltpu.VMEM((64, D), F32),
        pltpu.VMEM((64, D), F32),
        pltpu.VMEM((64, D), F32),
    ],
)


def _final_body(osad, Q0, Q1, x_pad, dis, deg1, u_r, v_r,
                c11, c12, c21, c22, cs12, cs21, du, dv,
                uqv, vqv,
                osu, q0u, q1u, xu, osv2, q0v, q1v, xv2,
                dsu, dsv, dgu, dgv,
                o11, o12, o21, o22, o512, o521, sem):
    c, s, w = _ids()
    pltpu.sync_copy(u_r.at[w], uqv)
    pltpu.sync_copy(v_r.at[w], vqv)

    def batch(b, _):
        iu = uqv.at[pl.ds(32 * b, 32)]
        iv = vqv.at[pl.ds(32 * b, 32)]
        descs = [
            pltpu.async_copy(osad.at[iu], osu, sem),
            pltpu.async_copy(Q0.at[iu], q0u, sem),
            pltpu.async_copy(Q1.at[iu], q1u, sem),
            pltpu.async_copy(x_pad.at[iu], xu, sem),
            pltpu.async_copy(osad.at[iv], osv2, sem),
            pltpu.async_copy(Q0.at[iv], q0v, sem),
            pltpu.async_copy(Q1.at[iv], q1v, sem),
            pltpu.async_copy(x_pad.at[iv], xv2, sem),
            pltpu.async_copy(dis.at[iu], dsu, sem),
            pltpu.async_copy(dis.at[iv], dsv, sem),
            pltpu.async_copy(deg1.at[iu], dgu, sem),
            pltpu.async_copy(deg1.at[iv], dgv, sem),
        ]
        for d_ in descs:
            d_.wait()
        qsl = pl.ds(128 * w + 32 * b, 32)
        pltpu.sync_copy(dgu, du.at[qsl])
        pltpu.sync_copy(dgv, dv.at[qsl])

        def row(r, _):
            bdu = _bc16(dsu, r)
            bdv = _bc16(dsv, r)
            bgu = _bc16(dgu, r)
            bgv = _bc16(dgv, r)
            inv_u = 1.0 / bdu
            inv_v = 1.0 / bdv
            for k in range(8):
                sl = pl.ds(k * 16, 16)
                os_u = osu[r, sl]
                os_v = osv2[r, sl]
                thu = bdu * (q0u[r, sl] + q1u[r, sl] + os_u)
                thv = bdv * (q0v[r, sl] + q1v[r, sl] + os_v)
                ohu = os_u * inv_u
                ohv = os_v * inv_v
                o11[r, sl] = ohu * ohv
                o12[r, sl] = ohu * thv
                o21[r, sl] = thu * ohv
                o22[r, sl] = (thu - bgu * xu[r, sl]) * (thv - bgv * xv2[r, sl])
                o512[r, sl] = ohu * thu
                o521[r, sl] = ohv * thv
            return 0
        lax.fori_loop(0, 32, row, 0)
        pltpu.sync_copy(o11, c11.at[qsl])
        pltpu.sync_copy(o12, c12.at[qsl])
        pltpu.sync_copy(o21, c21.at[qsl])
        pltpu.sync_copy(o22, c22.at[qsl])
        pltpu.sync_copy(o512, cs12.at[qsl])
        pltpu.sync_copy(o521, cs21.at[qsl])
        return 0
    lax.fori_loop(0, 4, batch, 0)


_ph_final = pl.kernel(
    _final_body,
    out_type=tuple([jax.ShapeDtypeStruct((NQ, D), F32)] * 6
                   + [jax.ShapeDtypeStruct((NQ,), F32)] * 2),
    mesh=MESH,
    scratch_types=(
        [pltpu.VMEM((128,), I32)] * 2
        + [pltpu.VMEM((32, D), F32)] * 8
        + [pltpu.VMEM((32,), F32)] * 4
        + [pltpu.VMEM((32, D), F32)] * 6
        + [pltpu.SemaphoreType.DMA]
    ),
)


def kernel(x, edges, edge_index):
    src = edge_index[0].astype(I32)
    dst = edge_index[1].astype(I32)
    padv = SENT + (jnp.arange(EP - E, dtype=I32) % (NP - N))
    srcp = jnp.concatenate([src, padv])
    dstp = jnp.concatenate([dst, padv])
    srcr32 = srcp.reshape(32, 79, 128)
    dstr32 = dstp.reshape(32, 79, 128)
    nodes_r = jnp.concatenate([edges[0], edges[1]]).astype(I32).reshape(32, 2, 128)
    x_pad = jnp.pad(x, ((0, NP - N), (0, 0)))
    u_r = edges[0].astype(I32).reshape(32, 128)
    v_r = edges[1].astype(I32).reshape(32, 128)

    markp = _ph_mark(nodes_r)
    n1p = _ph_hop(markp, srcr32, dstr32)
    n2p = _ph_hop(n1p, srcr32, dstr32)
    pk2r, degp = _ph_valid(markp, n1p, n2p, srcr32, dstr32)
    deg1, dis, xs = _ph_scale(degp, x_pad)
    P = _ph_spmm(xs, pk2r)
    osad = _ph_os(P, xs, dis)
    Q = _ph_spmm(osad, pk2r)
    Q0 = Q[:NP]
    Q1 = Q[NP:]
    outs = _ph_final(osad, Q0, Q1, x_pad, dis, deg1, u_r, v_r)
    return tuple(outs)


# speedup vs baseline: 1.0810x; 1.0810x over previous
import functools

import jax
import jax.numpy as jnp
from jax import lax
from jax.experimental import pallas as pl
from jax.experimental.pallas import tpu as pltpu
from jax.experimental.pallas import tpu_sc as plsc

N = 10000
D = 128
E = 320000
NQ = 4096
NP = 10240
SENT = 10000
EP = 323584
F32 = jnp.float32
I32 = jnp.int32

MESH = plsc.VectorSubcoreMesh(
    core_axis_name="c", subcore_axis_name="s", num_cores=2, num_subcores=16)

def _ids():
    c = lax.axis_index("c")
    s = lax.axis_index("s")
    return c, s, c * 16 + s


def _fill1(ref, n, val, dtype):
    v = jnp.full((16,), val, dtype)
    def body(i, _):
        ref[pl.ds(i * 16, 16)] = v
        return 0
    lax.fori_loop(0, n // 16, body, 0)


def _fill2(ref, rows, val, dtype):
    v = jnp.full((16,), val, dtype)
    def body(i, _):
        for k in range(8):
            ref[i, pl.ds(k * 16, 16)] = v
        return 0
    lax.fori_loop(0, rows, body, 0)


def _bc16(ref, i):
    j = (i // 16) * 16
    piece = ref[pl.ds(j, 16)]
    dn = lax.GatherDimensionNumbers(
        offset_dims=(), collapsed_slice_dims=(0,), start_index_map=(0,))
    return lax.gather(piece, jnp.full((16, 1), i - j, I32), dn,
                      slice_sizes=(1,),
                      mode=lax.GatherScatterMode.PROMISE_IN_BOUNDS)


def _rsqrt(d):
    h = d * (-0.5)
    i = lax.bitcast_convert_type(d, I32)
    i = jnp.int32(0x5F3759DF) - (i >> 1)
    y = lax.bitcast_convert_type(i, F32)
    for _ in range(4):
        y = y * (1.5 + h * y * y)
    return y


def _mark_body(nodes_r, out_hbm, mark_sh, zb, ones, idxv):
    c, s, _ = _ids()
    _fill1(zb, 640, 0, I32)
    _fill1(ones, 128, 1, I32)
    pltpu.sync_copy(zb, mark_sh.at[pl.ds(640 * s, 640)])
    plsc.subcore_barrier()
    w = c * 16 + s
    pltpu.sync_copy(nodes_r.at[w], idxv)
    for b in range(2):
        pltpu.sync_copy(ones, mark_sh.at[idxv.at[b]], add=True)
    plsc.subcore_barrier()
    pltpu.sync_copy(mark_sh.at[pl.ds(640 * s, 640)],
                    out_hbm.at[pl.ds(c * NP + 640 * s, 640)])


_ph_mark = pl.kernel(
    _mark_body,
    out_type=jax.ShapeDtypeStruct((2 * NP,), I32),
    mesh=MESH,
    scratch_types=[
        pltpu.VMEM_SHARED((NP,), I32),
        pltpu.VMEM((640,), I32),
        pltpu.VMEM((128,), I32),
        pltpu.VMEM((2, 128), I32),
    ],
)


def _hop_body(prev, srcr, dstr, out_hbm,
              cur_sh, nxt_sh, zb, av, bv, cv, srcv, dstv, g0, g1,
              sem0, sem1):
    c, s, w = _ids()
    base = 640 * s
    _fill1(zb, 640, 0, I32)
    pltpu.sync_copy(prev.at[pl.ds(base, 640)], av)
    pltpu.sync_copy(prev.at[pl.ds(NP + base, 640)], bv)
    def thr(i, _):
        sl = pl.ds(i * 16, 16)
        cv[sl] = jnp.where(av[sl] + bv[sl] > 0, 1, 0).astype(I32)
        return 0
    lax.fori_loop(0, 40, thr, 0)
    pltpu.sync_copy(cv, cur_sh.at[pl.ds(base, 640)])
    pltpu.sync_copy(zb, nxt_sh.at[pl.ds(base, 640)])
    plsc.subcore_barrier()
    pltpu.sync_copy(srcr.at[w], srcv)
    pltpu.sync_copy(dstr.at[w], dstv)

    pltpu.async_copy(cur_sh.at[dstv.at[0]], g0, sem0)
    def body(jj, _):
        j0 = 2 * jj
        j1 = j0 + 1
        @pl.when(j1 < 79)
        def _():
            pltpu.async_copy(cur_sh.at[dstv.at[j1]], g1, sem1)
        pltpu.make_async_copy(cur_sh.at[dstv.at[j0]], g0, sem0).wait()
        pltpu.sync_copy(g0, nxt_sh.at[srcv.at[j0]], add=True)
        @pl.when(j0 + 2 < 79)
        def _():
            pltpu.async_copy(cur_sh.at[dstv.at[j0 + 2]], g0, sem0)
        @pl.when(j1 < 79)
        def _():
            pltpu.make_async_copy(cur_sh.at[dstv.at[j1]], g1, sem1).wait()
            pltpu.sync_copy(g1, nxt_sh.at[srcv.at[j1]], add=True)
        return 0
    lax.fori_loop(0, 40, body, 0)
    plsc.subcore_barrier()
    pltpu.sync_copy(nxt_sh.at[pl.ds(base, 640)],
                    out_hbm.at[pl.ds(c * NP + base, 640)])


_ph_hop = pl.kernel(
    _hop_body,
    out_type=jax.ShapeDtypeStruct((2 * NP,), I32),
    mesh=MESH,
    scratch_types=[
        pltpu.VMEM_SHARED((NP,), I32),
        pltpu.VMEM_SHARED((NP,), I32),
        pltpu.VMEM((640,), I32),
        pltpu.VMEM((640,), I32),
        pltpu.VMEM((640,), I32),
        pltpu.VMEM((640,), I32),
        pltpu.VMEM((79, 128), I32),
        pltpu.VMEM((79, 128), I32),
        pltpu.VMEM((128,), I32),
        pltpu.VMEM((128,), I32),
        pltpu.SemaphoreType.DMA,
        pltpu.SemaphoreType.DMA,
    ],
)


def _valid_body(markp, n1p, n2p, srcr, dstr, pk2r, degp,
                reach_sh, deg_sh, zbf, m0, m1, a1, b1, a2, b2, cv,
                srcv, dstv, pkv, rs, rd, rs1, rd1, vf,
                sem, sem1):
    c, s, w = _ids()
    base = 640 * s
    _fill1(zbf, 640, 0.0, F32)
    pltpu.sync_copy(markp.at[pl.ds(base, 640)], m0)
    pltpu.sync_copy(markp.at[pl.ds(NP + base, 640)], m1)
    pltpu.sync_copy(n1p.at[pl.ds(base, 640)], a1)
    pltpu.sync_copy(n1p.at[pl.ds(NP + base, 640)], b1)
    pltpu.sync_copy(n2p.at[pl.ds(base, 640)], a2)
    pltpu.sync_copy(n2p.at[pl.ds(NP + base, 640)], b2)
    def thr(i, _):
        sl = pl.ds(i * 16, 16)
        tot = m0[sl] + m1[sl] + a1[sl] + b1[sl] + a2[sl] + b2[sl]
        cv[sl] = jnp.where(tot > 0, 1, 0).astype(I32)
        return 0
    lax.fori_loop(0, 40, thr, 0)
    pltpu.sync_copy(cv, reach_sh.at[pl.ds(base, 640)])
    pltpu.sync_copy(zbf, deg_sh.at[pl.ds(base, 640)])
    plsc.subcore_barrier()
    pltpu.sync_copy(srcr.at[w], srcv)
    pltpu.sync_copy(dstr.at[w], dstv)

    def issue(j, rsb, rdb, semb):
        pltpu.async_copy(reach_sh.at[srcv.at[j]], rsb, semb)
        pltpu.async_copy(reach_sh.at[dstv.at[j]], rdb, semb)

    def drain(j, rsb, rdb, semb):
        pltpu.make_async_copy(reach_sh.at[srcv.at[j]], rsb, semb).wait()
        pltpu.make_async_copy(reach_sh.at[dstv.at[j]], rdb, semb).wait()

    def compute(j, rsb, rdb):
        for k in range(8):
            sl = pl.ds(k * 16, 16)
            junk = lax.iota(I32, 16) + jnp.int32(SENT + 16 * k)
            junk_pk = junk | (junk << 16)
            valid = rsb[sl] & rdb[sl]
            ok = valid > 0
            packed = srcv[j, sl] | (dstv[j, sl] << 16)
            pkv[j, sl] = jnp.where(ok, packed, junk_pk)
            vf[sl] = valid.astype(F32)
        pltpu.sync_copy(vf, deg_sh.at[srcv.at[j]], add=True)

    def body(jj, _):
        j0 = 2 * jj
        j1 = j0 + 1
        @pl.when(j1 < 79)
        def _():
            issue(j1, rs1, rd1, sem1)
        drain(j0, rs, rd, sem)
        compute(j0, rs, rd)
        @pl.when(j0 + 2 < 79)
        def _():
            issue(j0 + 2, rs, rd, sem)
        @pl.when(j1 < 79)
        def _():
            drain(j1, rs1, rd1, sem1)
            compute(j1, rs1, rd1)
        return 0
    issue(0, rs, rd, sem)
    lax.fori_loop(0, 40, body, 0)
    pltpu.sync_copy(pkv, pk2r.at[w])
    plsc.subcore_barrier()
    pltpu.sync_copy(deg_sh.at[pl.ds(640 * s, 640)],
                    degp.at[pl.ds(c * NP + 640 * s, 640)])


_ph_valid = pl.kernel(
    _valid_body,
    out_type=(
        jax.ShapeDtypeStruct((32, 79, 128), I32),
        jax.ShapeDtypeStruct((2 * NP,), F32),
    ),
    mesh=MESH,
    scratch_types=[
        pltpu.VMEM_SHARED((NP,), I32),
        pltpu.VMEM_SHARED((NP,), F32),
        pltpu.VMEM((640,), F32),
        pltpu.VMEM((640,), I32),
        pltpu.VMEM((640,), I32),
        pltpu.VMEM((640,), I32),
        pltpu.VMEM((640,), I32),
        pltpu.VMEM((640,), I32),
        pltpu.VMEM((640,), I32),
        pltpu.VMEM((640,), I32),
        pltpu.VMEM((79, 128), I32),
        pltpu.VMEM((79, 128), I32),
        pltpu.VMEM((79, 128), I32),
        pltpu.VMEM((128,), I32),
        pltpu.VMEM((128,), I32),
        pltpu.VMEM((128,), I32),
        pltpu.VMEM((128,), I32),
        pltpu.VMEM((128,), F32),
        pltpu.SemaphoreType.DMA,
        pltpu.SemaphoreType.DMA,
    ],
)


def _scale_body(degp, x_pad, deg1, dis, xs,
                dp0v, dp1v, d1v, disv, xv, xsv):
    c, s, w = _ids()
    base = 320 * w
    pltpu.sync_copy(degp.at[pl.ds(base, 320)], dp0v)
    pltpu.sync_copy(degp.at[pl.ds(NP + base, 320)], dp1v)

    def body(i, _):
        sl = pl.ds(i * 16, 16)
        d1 = dp0v[sl] + dp1v[sl]
        d1v[sl] = d1
        disv[sl] = _rsqrt(d1 + 1.0)
        return 0
    lax.fori_loop(0, 20, body, 0)
    pltpu.sync_copy(d1v, deg1.at[pl.ds(base, 320)])
    pltpu.sync_copy(disv, dis.at[pl.ds(base, 320)])

    def chunk(cc, _):
        pltpu.sync_copy(x_pad.at[pl.ds(base + 64 * cc, 64)], xv)
        def row(r, _):
            b = _bc16(disv, 64 * cc + r)
            for k in range(8):
                sl = pl.ds(k * 16, 16)
                xsv[r, sl] = b * xv[r, sl]
            return 0
        lax.fori_loop(0, 64, row, 0)
        pltpu.sync_copy(xsv, xs.at[pl.ds(base + 64 * cc, 64)])
        return 0
    lax.fori_loop(0, 5, chunk, 0)


_ph_scale = pl.kernel(
    _scale_body,
    out_type=(
        jax.ShapeDtypeStruct((NP,), F32),
        jax.ShapeDtypeStruct((NP,), F32),
        jax.ShapeDtypeStruct((NP, D), F32),
    ),
    mesh=MESH,
    scratch_types=[
        pltpu.VMEM((320,), F32),
        pltpu.VMEM((320,), F32),
        pltpu.VMEM((320,), F32),
        pltpu.VMEM((320,), F32),
        pltpu.VMEM((64, D), F32),
        pltpu.VMEM((64, D), F32),
    ],
)


def _spmm_body(T, pkr, P, acc_sh, zb2, pk, sc0, dc0, sc1, dc1,
               rv0, rv1, sg0a, sg0b, sg1a, sg1b):
    c, s, w = _ids()
    _fill2(zb2, 32, 0.0, F32)
    def zc(cc, _):
        pltpu.sync_copy(zb2, acc_sh.at[pl.ds(640 * s + 32 * cc, 32)])
        return 0
    lax.fori_loop(0, 20, zc, 0)
    plsc.subcore_barrier()
    pltpu.sync_copy(pkr.at[w], pk)

    def issue(j, sb, db, rvb, sema, semb):
        for k in range(8):
            sl = pl.ds(k * 16, 16)
            pkd = pk[j, sl]
            sb[sl] = pkd & jnp.int32(0xFFFF)
            db[sl] = lax.shift_right_logical(pkd, 16)
        pltpu.async_copy(T.at[db.at[pl.ds(0, 64)]], rvb.at[pl.ds(0, 64)], sema)
        pltpu.async_copy(T.at[db.at[pl.ds(64, 64)]], rvb.at[pl.ds(64, 64)],
                         semb)

    def drain(db, rvb, sema, semb):
        pltpu.make_async_copy(T.at[db.at[pl.ds(0, 64)]],
                              rvb.at[pl.ds(0, 64)], sema).wait()
        pltpu.make_async_copy(T.at[db.at[pl.ds(64, 64)]],
                              rvb.at[pl.ds(64, 64)], semb).wait()

    issue(0, sc0, dc0, rv0, sg0a, sg0b)
    def body(jj, _):
        j0 = 2 * jj
        j1 = j0 + 1
        @pl.when(j1 < 79)
        def _():
            issue(j1, sc1, dc1, rv1, sg1a, sg1b)
        drain(dc0, rv0, sg0a, sg0b)
        pltpu.sync_copy(rv0, acc_sh.at[sc0], add=True)
        @pl.when(j0 + 2 < 79)
        def _():
            issue(j0 + 2, sc0, dc0, rv0, sg0a, sg0b)
        @pl.when(j1 < 79)
        def _():
            drain(dc1, rv1, sg1a, sg1b)
            pltpu.sync_copy(rv1, acc_sh.at[sc1], add=True)
        return 0
    lax.fori_loop(0, 40, body, 0)
    plsc.subcore_barrier()
    pltpu.sync_copy(acc_sh.at[pl.ds(640 * s, 640)],
                    P.at[pl.ds(c * NP + 640 * s, 640)])


_ph_spmm = pl.kernel(
    _spmm_body,
    out_type=jax.ShapeDtypeStruct((2 * NP, D), F32),
    mesh=MESH,
    scratch_types=(
        [pltpu.VMEM_SHARED((NP, D), F32),
         pltpu.VMEM((32, D), F32),
         pltpu.VMEM((79, 128), I32)]
        + [pltpu.VMEM((128,), I32)] * 4
        + [pltpu.VMEM((128, D), F32)] * 2
        + [pltpu.SemaphoreType.DMA] * 4
    ),
)


def _os_body(P, xs, dis, osad, disv, p0v, p1v, xsv, ov):
    c, s, w = _ids()
    base = 320 * w
    pltpu.sync_copy(dis.at[pl.ds(base, 320)], disv)

    def chunk(cc, _):
        sl64 = pl.ds(base + 64 * cc, 64)
        pltpu.sync_copy(P.at[sl64], p0v)
        pltpu.sync_copy(P.at[pl.ds(NP + base + 64 * cc, 64)], p1v)
        pltpu.sync_copy(xs.at[sl64], xsv)
        def row(r, _):
            b = _bc16(disv, 64 * cc + r)
            b2 = b * b
            for k in range(8):
                sl = pl.ds(k * 16, 16)
                ov[r, sl] = b2 * (p0v[r, sl] + p1v[r, sl] + xsv[r, sl])
            return 0
        lax.fori_loop(0, 64, row, 0)
        pltpu.sync_copy(ov, osad.at[sl64])
        return 0
    lax.fori_loop(0, 5, chunk, 0)


_ph_os = pl.kernel(
    _os_body,
    out_type=jax.ShapeDtypeStruct((NP, D), F32),
    mesh=MESH,
    scratch_types=[
        pltpu.VMEM((320,), F32),
        pltpu.VMEM((64, D), F32),
        pltpu.VMEM((64, D), F32),
        pltpu.VMEM((64, D), F32),
        pltpu.VMEM((64, D), F32),
    ],
)


def _final_body(osad, Q0, Q1, x_pad, dis, deg1, u_r, v_r,
                c11, c12, c21, c22, cs12, cs21, du, dv,
                uqv, vqv,
                osu, q0u, q1u, xu, osv2, q0v, q1v, xv2,
                dsu, dsv, dgu, dgv,
                o11, o12, o21, o22, o512, o521, sem):
    c, s, w = _ids()
    pltpu.sync_copy(u_r.at[w], uqv)
    pltpu.sync_copy(v_r.at[w], vqv)

    def batch(b, _):
        iu = uqv.at[pl.ds(32 * b, 32)]
        iv = vqv.at[pl.ds(32 * b, 32)]
        descs = [
            pltpu.async_copy(osad.at[iu], osu, sem),
            pltpu.async_copy(Q0.at[iu], q0u, sem),
            pltpu.async_copy(Q1.at[iu], q1u, sem),
            pltpu.async_copy(x_pad.at[iu], xu, sem),
            pltpu.async_copy(osad.at[iv], osv2, sem),
            pltpu.async_copy(Q0.at[iv], q0v, sem),
            pltpu.async_copy(Q1.at[iv], q1v, sem),
            pltpu.async_copy(x_pad.at[iv], xv2, sem),
            pltpu.async_copy(dis.at[iu], dsu, sem),
            pltpu.async_copy(dis.at[iv], dsv, sem),
            pltpu.async_copy(deg1.at[iu], dgu, sem),
            pltpu.async_copy(deg1.at[iv], dgv, sem),
        ]
        for d_ in descs:
            d_.wait()
        qsl = pl.ds(128 * w + 32 * b, 32)
        pltpu.sync_copy(dgu, du.at[qsl])
        pltpu.sync_copy(dgv, dv.at[qsl])

        def row(r, _):
            bdu = _bc16(dsu, r)
            bdv = _bc16(dsv, r)
            bgu = _bc16(dgu, r)
            bgv = _bc16(dgv, r)
            inv_u = 1.0 / bdu
            inv_v = 1.0 / bdv
            for k in range(8):
                sl = pl.ds(k * 16, 16)
                os_u = osu[r, sl]
                os_v = osv2[r, sl]
                thu = bdu * (q0u[r, sl] + q1u[r, sl] + os_u)
                thv = bdv * (q0v[r, sl] + q1v[r, sl] + os_v)
                ohu = os_u * inv_u
                ohv = os_v * inv_v
                o11[r, sl] = ohu * ohv
                o12[r, sl] = ohu * thv
                o21[r, sl] = thu * ohv
                o22[r, sl] = (thu - bgu * xu[r, sl]) * (thv - bgv * xv2[r, sl])
                o512[r, sl] = ohu * thu
                o521[r, sl] = ohv * thv
            return 0
        lax.fori_loop(0, 32, row, 0)
        pltpu.sync_copy(o11, c11.at[qsl])
        pltpu.sync_copy(o12, c12.at[qsl])
        pltpu.sync_copy(o21, c21.at[qsl])
        pltpu.sync_copy(o22, c22.at[qsl])
        pltpu.sync_copy(o512, cs12.at[qsl])
        pltpu.sync_copy(o521, cs21.at[qsl])
        return 0
    lax.fori_loop(0, 4, batch, 0)


_ph_final = pl.kernel(
    _final_body,
    out_type=tuple([jax.ShapeDtypeStruct((NQ, D), F32)] * 6
                   + [jax.ShapeDtypeStruct((NQ,), F32)] * 2),
    mesh=MESH,
    scratch_types=(
        [pltpu.VMEM((128,), I32)] * 2
        + [pltpu.VMEM((32, D), F32)] * 8
        + [pltpu.VMEM((32,), F32)] * 4
        + [pltpu.VMEM((32, D), F32)] * 6
        + [pltpu.SemaphoreType.DMA]
    ),
)


def kernel(x, edges, edge_index):
    src = edge_index[0].astype(I32)
    dst = edge_index[1].astype(I32)
    padv = SENT + (jnp.arange(EP - E, dtype=I32) % (NP - N))
    srcp = jnp.concatenate([src, padv])
    dstp = jnp.concatenate([dst, padv])
    srcr32 = srcp.reshape(32, 79, 128)
    dstr32 = dstp.reshape(32, 79, 128)
    nodes_r = jnp.concatenate([edges[0], edges[1]]).astype(I32).reshape(32, 2, 128)
    x_pad = jnp.pad(x, ((0, NP - N), (0, 0)))
    u_r = edges[0].astype(I32).reshape(32, 128)
    v_r = edges[1].astype(I32).reshape(32, 128)

    markp = _ph_mark(nodes_r)
    n1p = _ph_hop(markp, srcr32, dstr32)
    n2p = _ph_hop(n1p, srcr32, dstr32)
    pk2r, degp = _ph_valid(markp, n1p, n2p, srcr32, dstr32)
    deg1, dis, xs = _ph_scale(degp, x_pad)
    P = _ph_spmm(xs, pk2r)
    osad = _ph_os(P, xs, dis)
    Q = _ph_spmm(osad, pk2r)
    Q0 = Q[:NP]
    Q1 = Q[NP:]
    outs = _ph_final(osad, Q0, Q1, x_pad, dis, deg1, u_r, v_r)
    return tuple(outs)


# speedup vs baseline: 1.0950x; 1.0130x over previous
import functools

import jax
import jax.numpy as jnp
from jax import lax
from jax.experimental import pallas as pl
from jax.experimental.pallas import tpu as pltpu
from jax.experimental.pallas import tpu_sc as plsc

N = 10000
D = 128
E = 320000
NQ = 4096
NP = 10240
SENT = 10000
EP = 323584
F32 = jnp.float32
I32 = jnp.int32

MESH = plsc.VectorSubcoreMesh(
    core_axis_name="c", subcore_axis_name="s", num_cores=2, num_subcores=16)

def _ids():
    c = lax.axis_index("c")
    s = lax.axis_index("s")
    return c, s, c * 16 + s


def _fill1(ref, n, val, dtype):
    v = jnp.full((16,), val, dtype)
    def body(i, _):
        ref[pl.ds(i * 16, 16)] = v
        return 0
    lax.fori_loop(0, n // 16, body, 0)


def _fill2(ref, rows, val, dtype):
    v = jnp.full((16,), val, dtype)
    def body(i, _):
        for k in range(8):
            ref[i, pl.ds(k * 16, 16)] = v
        return 0
    lax.fori_loop(0, rows, body, 0)


def _bc16(ref, i):
    j = (i // 16) * 16
    piece = ref[pl.ds(j, 16)]
    dn = lax.GatherDimensionNumbers(
        offset_dims=(), collapsed_slice_dims=(0,), start_index_map=(0,))
    return lax.gather(piece, jnp.full((16, 1), i - j, I32), dn,
                      slice_sizes=(1,),
                      mode=lax.GatherScatterMode.PROMISE_IN_BOUNDS)


def _rsqrt(d):
    h = d * (-0.5)
    i = lax.bitcast_convert_type(d, I32)
    i = jnp.int32(0x5F3759DF) - (i >> 1)
    y = lax.bitcast_convert_type(i, F32)
    for _ in range(4):
        y = y * (1.5 + h * y * y)
    return y


def _hop_body(first, prev, srcr, dstr, out_hbm, markv,
              cur_sh, nxt_sh, zb, av, bv, cv, srcv, dstv, g0, g1, t0, t1,
              sem0, sem1):
    c, s, w = _ids()
    base = 640 * s
    _fill1(zb, 640, 0, I32)
    pltpu.sync_copy(zb, nxt_sh.at[pl.ds(base, 640)])
    if first:
        _fill1(av, 128, 1, I32)
        pltpu.sync_copy(zb, cur_sh.at[pl.ds(base, 640)])
        plsc.subcore_barrier()
        pltpu.sync_copy(prev.at[s], dstv.at[pl.ds(0, 4)])
        for b in range(4):
            pltpu.sync_copy(av.at[pl.ds(0, 128)],
                            cur_sh.at[dstv.at[b]], add=True)
    else:
        pltpu.sync_copy(prev.at[pl.ds(base, 640)], av)
        pltpu.sync_copy(prev.at[pl.ds(NP + base, 640)], bv)
        def thr(i, _):
            sl = pl.ds(i * 16, 16)
            cv[sl] = jnp.where(av[sl] + bv[sl] > 0, 1, 0).astype(I32)
            return 0
        lax.fori_loop(0, 40, thr, 0)
        pltpu.sync_copy(cv, cur_sh.at[pl.ds(base, 640)])
    plsc.subcore_barrier()
    pltpu.sync_copy(srcr.at[w], srcv)
    pltpu.sync_copy(dstr.at[w], dstv)

    def scat(j, gb, tb):
        if first:
            for k in range(8):
                sl = pl.ds(k * 16, 16)
                tb[sl] = jnp.where(gb[sl] > 0, 1, 0).astype(I32)
            pltpu.sync_copy(tb, nxt_sh.at[srcv.at[j]], add=True)
        else:
            pltpu.sync_copy(gb, nxt_sh.at[srcv.at[j]], add=True)

    pltpu.async_copy(cur_sh.at[dstv.at[0]], g0, sem0)
    def body(jj, _):
        j0 = 2 * jj
        j1 = j0 + 1
        @pl.when(j1 < 79)
        def _():
            pltpu.async_copy(cur_sh.at[dstv.at[j1]], g1, sem1)
        pltpu.make_async_copy(cur_sh.at[dstv.at[j0]], g0, sem0).wait()
        scat(j0, g0, t0)
        @pl.when(j0 + 2 < 79)
        def _():
            pltpu.async_copy(cur_sh.at[dstv.at[j0 + 2]], g0, sem0)
        @pl.when(j1 < 79)
        def _():
            pltpu.make_async_copy(cur_sh.at[dstv.at[j1]], g1, sem1).wait()
            scat(j1, g1, t1)
        return 0
    lax.fori_loop(0, 40, body, 0)
    plsc.subcore_barrier()
    pltpu.sync_copy(nxt_sh.at[pl.ds(base, 640)],
                    out_hbm.at[pl.ds(c * NP + base, 640)])
    if first:
        @pl.when(c == 0)
        def _():
            pltpu.sync_copy(cur_sh.at[pl.ds(base, 640)],
                            markv.at[pl.ds(base, 640)])


def _mk_hop(first):
    outs = [jax.ShapeDtypeStruct((2 * NP,), I32)]
    if first:
        outs.append(jax.ShapeDtypeStruct((NP,), I32))
    body = (functools.partial(_hop_body, first) if first else
            (lambda prev, srcr, dstr, out_hbm, *sc:
             _hop_body(False, prev, srcr, dstr, out_hbm, None, *sc)))
    return pl.kernel(
        body,
        out_type=tuple(outs) if first else outs[0],
        mesh=MESH,
        scratch_types=[
            pltpu.VMEM_SHARED((NP,), I32),
            pltpu.VMEM_SHARED((NP,), I32),
            pltpu.VMEM((640,), I32),
            pltpu.VMEM((640,), I32),
            pltpu.VMEM((640,), I32),
            pltpu.VMEM((640,), I32),
            pltpu.VMEM((79, 128), I32),
            pltpu.VMEM((79, 128), I32),
            pltpu.VMEM((128,), I32),
            pltpu.VMEM((128,), I32),
            pltpu.VMEM((128,), I32),
            pltpu.VMEM((128,), I32),
            pltpu.SemaphoreType.DMA,
            pltpu.SemaphoreType.DMA,
        ],
    )


_ph_hop1 = _mk_hop(True)
_ph_hop2 = _mk_hop(False)


def _valid_body(markv, n1p, n2p, srcr, dstr, pk2r, degp,
                reach_sh, deg_sh, zbf, m0, a1, b1, a2, b2, cv,
                srcv, dstv, pkv, rs, rd, rs1, rd1, vf,
                sem, sem1):
    c, s, w = _ids()
    base = 640 * s
    _fill1(zbf, 640, 0.0, F32)
    pltpu.sync_copy(markv.at[pl.ds(base, 640)], m0)
    pltpu.sync_copy(n1p.at[pl.ds(base, 640)], a1)
    pltpu.sync_copy(n1p.at[pl.ds(NP + base, 640)], b1)
    pltpu.sync_copy(n2p.at[pl.ds(base, 640)], a2)
    pltpu.sync_copy(n2p.at[pl.ds(NP + base, 640)], b2)
    def thr(i, _):
        sl = pl.ds(i * 16, 16)
        tot = m0[sl] + a1[sl] + b1[sl] + a2[sl] + b2[sl]
        cv[sl] = jnp.where(tot > 0, 1, 0).astype(I32)
        return 0
    lax.fori_loop(0, 40, thr, 0)
    pltpu.sync_copy(cv, reach_sh.at[pl.ds(base, 640)])
    pltpu.sync_copy(zbf, deg_sh.at[pl.ds(base, 640)])
    plsc.subcore_barrier()
    pltpu.sync_copy(srcr.at[w], srcv)
    pltpu.sync_copy(dstr.at[w], dstv)

    def issue(j, rsb, rdb, semb):
        pltpu.async_copy(reach_sh.at[srcv.at[j]], rsb, semb)
        pltpu.async_copy(reach_sh.at[dstv.at[j]], rdb, semb)

    def drain(j, rsb, rdb, semb):
        pltpu.make_async_copy(reach_sh.at[srcv.at[j]], rsb, semb).wait()
        pltpu.make_async_copy(reach_sh.at[dstv.at[j]], rdb, semb).wait()

    def compute(j, rsb, rdb):
        for k in range(8):
            sl = pl.ds(k * 16, 16)
            junk = lax.iota(I32, 16) + jnp.int32(SENT + 16 * k)
            junk_pk = junk | (junk << 16)
            valid = rsb[sl] & rdb[sl]
            ok = valid > 0
            packed = srcv[j, sl] | (dstv[j, sl] << 16)
            pkv[j, sl] = jnp.where(ok, packed, junk_pk)
            vf[sl] = valid.astype(F32)
        pltpu.sync_copy(vf, deg_sh.at[srcv.at[j]], add=True)

    def body(jj, _):
        j0 = 2 * jj
        j1 = j0 + 1
        @pl.when(j1 < 79)
        def _():
            issue(j1, rs1, rd1, sem1)
        drain(j0, rs, rd, sem)
        compute(j0, rs, rd)
        @pl.when(j0 + 2 < 79)
        def _():
            issue(j0 + 2, rs, rd, sem)
        @pl.when(j1 < 79)
        def _():
            drain(j1, rs1, rd1, sem1)
            compute(j1, rs1, rd1)
        return 0
    issue(0, rs, rd, sem)
    lax.fori_loop(0, 40, body, 0)
    pltpu.sync_copy(pkv, pk2r.at[w])
    plsc.subcore_barrier()
    pltpu.sync_copy(deg_sh.at[pl.ds(640 * s, 640)],
                    degp.at[pl.ds(c * NP + 640 * s, 640)])


_ph_valid = pl.kernel(
    _valid_body,
    out_type=(
        jax.ShapeDtypeStruct((32, 79, 128), I32),
        jax.ShapeDtypeStruct((2 * NP,), F32),
    ),
    mesh=MESH,
    scratch_types=[
        pltpu.VMEM_SHARED((NP,), I32),
        pltpu.VMEM_SHARED((NP,), F32),
        pltpu.VMEM((640,), F32),
        pltpu.VMEM((640,), I32),
        pltpu.VMEM((640,), I32),
        pltpu.VMEM((640,), I32),
        pltpu.VMEM((640,), I32),
        pltpu.VMEM((640,), I32),
        pltpu.VMEM((640,), I32),
        pltpu.VMEM((79, 128), I32),
        pltpu.VMEM((79, 128), I32),
        pltpu.VMEM((79, 128), I32),
        pltpu.VMEM((128,), I32),
        pltpu.VMEM((128,), I32),
        pltpu.VMEM((128,), I32),
        pltpu.VMEM((128,), I32),
        pltpu.VMEM((128,), F32),
        pltpu.SemaphoreType.DMA,
        pltpu.SemaphoreType.DMA,
    ],
)


def _scale_body(degp, x_pad, deg1, dis, xs,
                dp0v, dp1v, d1v, disv, xv, xsv):
    c, s, w = _ids()
    base = 320 * w
    pltpu.sync_copy(degp.at[pl.ds(base, 320)], dp0v)
    pltpu.sync_copy(degp.at[pl.ds(NP + base, 320)], dp1v)

    def body(i, _):
        sl = pl.ds(i * 16, 16)
        d1 = dp0v[sl] + dp1v[sl]
        d1v[sl] = d1
        disv[sl] = _rsqrt(d1 + 1.0)
        return 0
    lax.fori_loop(0, 20, body, 0)
    pltpu.sync_copy(d1v, deg1.at[pl.ds(base, 320)])
    pltpu.sync_copy(disv, dis.at[pl.ds(base, 320)])

    def chunk(cc, _):
        pltpu.sync_copy(x_pad.at[pl.ds(base + 64 * cc, 64)], xv)
        def row(r, _):
            b = _bc16(disv, 64 * cc + r)
            for k in range(8):
                sl = pl.ds(k * 16, 16)
                xsv[r, sl] = b * xv[r, sl]
            return 0
        lax.fori_loop(0, 64, row, 0)
        pltpu.sync_copy(xsv, xs.at[pl.ds(base + 64 * cc, 64)])
        return 0
    lax.fori_loop(0, 5, chunk, 0)


_ph_scale = pl.kernel(
    _scale_body,
    out_type=(
        jax.ShapeDtypeStruct((NP,), F32),
        jax.ShapeDtypeStruct((NP,), F32),
        jax.ShapeDtypeStruct((NP, D), F32),
    ),
    mesh=MESH,
    scratch_types=[
        pltpu.VMEM((320,), F32),
        pltpu.VMEM((320,), F32),
        pltpu.VMEM((320,), F32),
        pltpu.VMEM((320,), F32),
        pltpu.VMEM((64, D), F32),
        pltpu.VMEM((64, D), F32),
    ],
)


def _spmm_body(T, pkr, P, acc_sh, zb2, pk, sc0, dc0, sc1, dc1,
               rv0, rv1, sg0a, sg1a):
    c, s, w = _ids()
    _fill2(zb2, 32, 0.0, F32)
    def zc(cc, _):
        pltpu.sync_copy(zb2, acc_sh.at[pl.ds(640 * s + 32 * cc, 32)])
        return 0
    lax.fori_loop(0, 20, zc, 0)
    plsc.subcore_barrier()
    pltpu.sync_copy(pkr.at[w], pk)

    def issue(j, sb, db, rvb, sema):
        for k in range(8):
            sl = pl.ds(k * 16, 16)
            pkd = pk[j, sl]
            sb[sl] = pkd & jnp.int32(0xFFFF)
            db[sl] = lax.shift_right_logical(pkd, 16)
        pltpu.async_copy(T.at[db], rvb, sema)

    issue(0, sc0, dc0, rv0, sg0a)
    def body(jj, _):
        j0 = 2 * jj
        j1 = j0 + 1
        @pl.when(j1 < 79)
        def _():
            issue(j1, sc1, dc1, rv1, sg1a)
        pltpu.make_async_copy(T.at[dc0], rv0, sg0a).wait()
        pltpu.sync_copy(rv0, acc_sh.at[sc0], add=True)
        @pl.when(j0 + 2 < 79)
        def _():
            issue(j0 + 2, sc0, dc0, rv0, sg0a)
        @pl.when(j1 < 79)
        def _():
            pltpu.make_async_copy(T.at[dc1], rv1, sg1a).wait()
            pltpu.sync_copy(rv1, acc_sh.at[sc1], add=True)
        return 0
    lax.fori_loop(0, 40, body, 0)
    plsc.subcore_barrier()
    pltpu.sync_copy(acc_sh.at[pl.ds(640 * s, 640)],
                    P.at[pl.ds(c * NP + 640 * s, 640)])


_ph_spmm = pl.kernel(
    _spmm_body,
    out_type=jax.ShapeDtypeStruct((2 * NP, D), F32),
    mesh=MESH,
    scratch_types=(
        [pltpu.VMEM_SHARED((NP, D), F32),
         pltpu.VMEM((32, D), F32),
         pltpu.VMEM((79, 128), I32)]
        + [pltpu.VMEM((128,), I32)] * 4
        + [pltpu.VMEM((128, D), F32)] * 2
        + [pltpu.SemaphoreType.DMA] * 2
    ),
)


def _os_body(P, xs, dis, osad, disv, p0v, p1v, xsv, ov):
    c, s, w = _ids()
    base = 320 * w
    pltpu.sync_copy(dis.at[pl.ds(base, 320)], disv)

    def chunk(cc, _):
        sl64 = pl.ds(base + 64 * cc, 64)
        pltpu.sync_copy(P.at[sl64], p0v)
        pltpu.sync_copy(P.at[pl.ds(NP + base + 64 * cc, 64)], p1v)
        pltpu.sync_copy(xs.at[sl64], xsv)
        def row(r, _):
            b = _bc16(disv, 64 * cc + r)
            b2 = b * b
            for k in range(8):
                sl = pl.ds(k * 16, 16)
                ov[r, sl] = b2 * (p0v[r, sl] + p1v[r, sl] + xsv[r, sl])
            return 0
        lax.fori_loop(0, 64, row, 0)
        pltpu.sync_copy(ov, osad.at[sl64])
        return 0
    lax.fori_loop(0, 5, chunk, 0)


_ph_os = pl.kernel(
    _os_body,
    out_type=jax.ShapeDtypeStruct((NP, D), F32),
    mesh=MESH,
    scratch_types=[
        pltpu.VMEM((320,), F32),
        pltpu.VMEM((64, D), F32),
        pltpu.VMEM((64, D), F32),
        pltpu.VMEM((64, D), F32),
        pltpu.VMEM((64, D), F32),
    ],
)


def _final_body(osad, Q0, Q1, x_pad, dis, deg1, u_r, v_r,
                c11, c12, c21, c22, cs12, cs21, du, dv,
                uqv, vqv,
                osu, q0u, q1u, xu, osv2, q0v, q1v, xv2,
                dsu, dsv, dgu, dgv,
                o11, o12, o21, o22, o512, o521, sem):
    c, s, w = _ids()
    pltpu.sync_copy(u_r.at[w], uqv)
    pltpu.sync_copy(v_r.at[w], vqv)

    def batch(b, _):
        iu = uqv.at[pl.ds(32 * b, 32)]
        iv = vqv.at[pl.ds(32 * b, 32)]
        descs = [
            pltpu.async_copy(osad.at[iu], osu, sem),
            pltpu.async_copy(Q0.at[iu], q0u, sem),
            pltpu.async_copy(Q1.at[iu], q1u, sem),
            pltpu.async_copy(x_pad.at[iu], xu, sem),
            pltpu.async_copy(osad.at[iv], osv2, sem),
            pltpu.async_copy(Q0.at[iv], q0v, sem),
            pltpu.async_copy(Q1.at[iv], q1v, sem),
            pltpu.async_copy(x_pad.at[iv], xv2, sem),
            pltpu.async_copy(dis.at[iu], dsu, sem),
            pltpu.async_copy(dis.at[iv], dsv, sem),
            pltpu.async_copy(deg1.at[iu], dgu, sem),
            pltpu.async_copy(deg1.at[iv], dgv, sem),
        ]
        for d_ in descs:
            d_.wait()
        qsl = pl.ds(128 * w + 32 * b, 32)
        pltpu.sync_copy(dgu, du.at[qsl])
        pltpu.sync_copy(dgv, dv.at[qsl])

        def row(r, _):
            bdu = _bc16(dsu, r)
            bdv = _bc16(dsv, r)
            bgu = _bc16(dgu, r)
            bgv = _bc16(dgv, r)
            inv_u = 1.0 / bdu
            inv_v = 1.0 / bdv
            for k in range(8):
                sl = pl.ds(k * 16, 16)
                os_u = osu[r, sl]
                os_v = osv2[r, sl]
                thu = bdu * (q0u[r, sl] + q1u[r, sl] + os_u)
                thv = bdv * (q0v[r, sl] + q1v[r, sl] + os_v)
                ohu = os_u * inv_u
                ohv = os_v * inv_v
                o11[r, sl] = ohu * ohv
                o12[r, sl] = ohu * thv
                o21[r, sl] = thu * ohv
                o22[r, sl] = (thu - bgu * xu[r, sl]) * (thv - bgv * xv2[r, sl])
                o512[r, sl] = ohu * thu
                o521[r, sl] = ohv * thv
            return 0
        lax.fori_loop(0, 32, row, 0)
        pltpu.sync_copy(o11, c11.at[qsl])
        pltpu.sync_copy(o12, c12.at[qsl])
        pltpu.sync_copy(o21, c21.at[qsl])
        pltpu.sync_copy(o22, c22.at[qsl])
        pltpu.sync_copy(o512, cs12.at[qsl])
        pltpu.sync_copy(o521, cs21.at[qsl])
        return 0
    lax.fori_loop(0, 4, batch, 0)


_ph_final = pl.kernel(
    _final_body,
    out_type=tuple([jax.ShapeDtypeStruct((NQ, D), F32)] * 6
                   + [jax.ShapeDtypeStruct((NQ,), F32)] * 2),
    mesh=MESH,
    scratch_types=(
        [pltpu.VMEM((128,), I32)] * 2
        + [pltpu.VMEM((32, D), F32)] * 8
        + [pltpu.VMEM((32,), F32)] * 4
        + [pltpu.VMEM((32, D), F32)] * 6
        + [pltpu.SemaphoreType.DMA]
    ),
)


def kernel(x, edges, edge_index):
    src = edge_index[0].astype(I32)
    dst = edge_index[1].astype(I32)
    padv = SENT + (jnp.arange(EP - E, dtype=I32) % (NP - N))
    srcp = jnp.concatenate([src, padv])
    dstp = jnp.concatenate([dst, padv])
    srcr32 = srcp.reshape(32, 79, 128)
    dstr32 = dstp.reshape(32, 79, 128)
    nodes_r = jnp.concatenate([edges[0], edges[1]]).astype(I32).reshape(16, 4, 128)
    x_pad = jnp.pad(x, ((0, NP - N), (0, 0)))
    u_r = edges[0].astype(I32).reshape(32, 128)
    v_r = edges[1].astype(I32).reshape(32, 128)

    n1p, markv = _ph_hop1(nodes_r, srcr32, dstr32)
    n2p = _ph_hop2(n1p, srcr32, dstr32)
    pk2r, degp = _ph_valid(markv, n1p, n2p, srcr32, dstr32)
    deg1, dis, xs = _ph_scale(degp, x_pad)
    P = _ph_spmm(xs, pk2r)
    osad = _ph_os(P, xs, dis)
    Q = _ph_spmm(osad, pk2r)
    Q0 = Q[:NP]
    Q1 = Q[NP:]
    outs = _ph_final(osad, Q0, Q1, x_pad, dis, deg1, u_r, v_r)
    return tuple(outs)
